# baseline (speedup 1.0000x reference)
import functools

import jax
import jax.numpy as jnp
import numpy as np
from jax import lax
from jax.experimental import pallas as pl
from jax.experimental.pallas import tpu as pltpu
from jax.experimental.pallas import tpu_sc as plsc

B = 16384
F = 26
V = 1000
D = 50
DP = 128
DK = 64
PW = DK // 2
ROW_W = F * PW
ROW_WP = 896
ROWS = B * F

NC = 2
NS = 16
NW = NC * NS
BW = B // NW
CB = 8
CH = CB * F
NITER = BW // CB


NUNIT = F * 4
QB = B // 4
CI = 512
NCH = QB // CI


def _gather_body(tpk_hbm, xct_hbm, out_hbm, tb, idx0, idx1, stg0, stg1,
                 ts, is0, is1, ws0, ws1):
    wid = lax.axis_index("s") * NC + lax.axis_index("c")

    def out_slice(f, i0):
        r0 = pl.multiple_of(f * PW, PW)
        c0 = pl.multiple_of(i0, CI)
        return out_hbm.at[pl.ds(r0, PW), pl.ds(c0, CI)]

    def do_chunk(f, cc, i0, idx_v, stg, isem, wsem, prev_f, prev_i0):
        pltpu.async_copy(xct_hbm.at[f, cc, 0], idx_v, isem).wait()

        @pl.when(prev_i0 >= 0)
        def _():
            pltpu.make_async_copy(stg, out_slice(prev_f, prev_i0), wsem).wait()

        def grp(j, carry):
            rows = idx_v[pl.ds(j * 16, 16)]
            rbase = rows * PW
            for w in range(PW):
                vals = plsc.load_gather(tb, [rbase + w])
                stg[w, pl.ds(j * 16, 16)] = vals
            return carry

        lax.fori_loop(0, CI // 16, grp, 0, unroll=False)
        pltpu.async_copy(stg, out_slice(f, i0), wsem)

    def do_unit(s, carry):
        pf0, pi0, pf1, pi1 = carry
        u = wid + 32 * s
        f = u // 4
        q = u - 4 * f

        t0 = pl.multiple_of(f * (V * PW), V * PW)
        pltpu.async_copy(tpk_hbm.at[pl.ds(t0, V * PW)], tb, ts).wait()
        for c in range(NCH):
            cc = q * NCH + c
            i0 = cc * CI
            if c % 2 == 0:
                do_chunk(f, cc, i0, idx0, stg0, is0, ws0, pf0, pi0)
                pf0, pi0 = f, i0
            else:
                do_chunk(f, cc, i0, idx1, stg1, is1, ws1, pf1, pi1)
                pf1, pi1 = f, i0
        return pf0, pi0, pf1, pi1

    nu = jnp.where(wid < NUNIT - 3 * NW, 4, 3)
    carry = (jnp.int32(0), jnp.int32(-1), jnp.int32(0), jnp.int32(-1))
    pf0, pi0, pf1, pi1 = lax.fori_loop(0, nu, do_unit, carry, unroll=False)

    @pl.when(pi0 >= 0)
    def _():
        pltpu.make_async_copy(stg0, out_slice(pf0, pi0), ws0).wait()

    @pl.when(pi1 >= 0)
    def _():
        pltpu.make_async_copy(stg1, out_slice(pf1, pi1), ws1).wait()


@jax.jit
def _gather(tpk, xct):
    mesh = plsc.VectorSubcoreMesh(core_axis_name="c", subcore_axis_name="s")
    kern = pl.kernel(
        _gather_body,
        out_type=jax.ShapeDtypeStruct((ROW_W, B), jnp.int32),
        mesh=mesh,
        scratch_types=[
            pltpu.VMEM((V * PW,), jnp.int32),
            pltpu.VMEM((CI,), jnp.int32),
            pltpu.VMEM((CI,), jnp.int32),
            pltpu.VMEM((PW, CI), jnp.int32),
            pltpu.VMEM((PW, CI), jnp.int32),
            pltpu.SemaphoreType.DMA,
            pltpu.SemaphoreType.DMA,
            pltpu.SemaphoreType.DMA,
            pltpu.SemaphoreType.DMA,
            pltpu.SemaphoreType.DMA,
        ],
        compiler_params=pltpu.CompilerParams(needs_layout_passes=False),
    )
    return kern(tpk, xct)


_CHUNK = 512
_NSTEP = B // _CHUNK


def _mlp_body(x_ref, wa_ref, wb_ref, b1_ref, w2_ref, b2_ref, w3_ref, b3_ref,
              g2_ref, be2_ref, g3_ref, be3_ref, out_ref,
              h1_scr, s1_scr, ss1_scr):
    i = pl.program_id(0)

    @pl.when(i == 0)
    def _():
        s1_scr[...] = jnp.zeros_like(s1_scr)
        ss1_scr[...] = jnp.zeros_like(ss1_scr)

    xw = x_ref[...]
    xa = lax.bitcast_convert_type(
        lax.shift_left(xw, 16), jnp.float32).astype(jnp.bfloat16)
    xb = lax.bitcast_convert_type(
        jnp.bitwise_and(xw, jnp.int32(-65536)), jnp.float32).astype(jnp.bfloat16)
    dn = (((0,), (0,)), ((), ()))
    h = (lax.dot_general(xa, wa_ref[...], dn, preferred_element_type=jnp.float32)
         + lax.dot_general(xb, wb_ref[...], dn, preferred_element_type=jnp.float32))
    h = jnp.maximum(h + b1_ref[...], 0.0)
    h1_scr[pl.ds(i * _CHUNK, _CHUNK), :] = h
    s1_scr[...] += jnp.sum(h, axis=0, keepdims=True)
    ss1_scr[...] += jnp.sum(h * h, axis=0, keepdims=True)

    @pl.when(i == _NSTEP - 1)
    def _():
        m1 = s1_scr[...] * (1.0 / B)
        v1 = ss1_scr[...] * (1.0 / B) - m1 * m1
        a1 = g2_ref[...] * lax.rsqrt(v1 + 1e-5)
        c1 = be2_ref[...] - m1 * a1
        h1 = h1_scr[...] * a1 + c1
        h2 = jnp.dot(h1, w2_ref[...], preferred_element_type=jnp.float32)
        h2 = jnp.maximum(h2 + b2_ref[...], 0.0)
        m2 = jnp.mean(h2, axis=0, keepdims=True)
        v2 = jnp.mean(h2 * h2, axis=0, keepdims=True) - m2 * m2
        a2 = g3_ref[...] * lax.rsqrt(v2 + 1e-5)
        c2 = be3_ref[...] - m2 * a2
        h2n = h2 * a2 + c2
        z = jnp.dot(h2n, w3_ref[...], preferred_element_type=jnp.float32)
        z = z + b3_ref[...]
        out_ref[...] = jax.nn.sigmoid(z)


@jax.jit
def _mlp(x, Wa, Wb, b1, W2, b2, W3, b3, g2, be2, g3, be3):
    const = lambda shape: pl.BlockSpec(shape, lambda i: (0, 0))
    return pl.pallas_call(
        _mlp_body,
        grid=(_NSTEP,),
        in_specs=[
            pl.BlockSpec((ROW_W, _CHUNK), lambda i: (0, i)),
            const((ROW_W, 150)),
            const((ROW_W, 150)),
            const((1, 150)),
            const((150, 75)),
            const((1, 75)),
            const((75, 1)),
            const((1, 1)),
            const((1, 150)),
            const((1, 150)),
            const((1, 75)),
            const((1, 75)),
        ],
        out_specs=pl.BlockSpec((B, 1), lambda i: (0, 0)),
        out_shape=jax.ShapeDtypeStruct((B, 1), jnp.float32),
        scratch_shapes=[
            pltpu.VMEM((B, 150), jnp.float32),
            pltpu.VMEM((1, 150), jnp.float32),
            pltpu.VMEM((1, 150), jnp.float32),
        ],
        compiler_params=pltpu.CompilerParams(
            dimension_semantics=("arbitrary",),
            vmem_limit_bytes=100 * 1024 * 1024,
        ),
    )(x, Wa, Wb, b1, W2, b2, W3, b3, g2, be2, g3, be3)


def _half_weights(W1):
    w = np.arange(ROW_W)
    f = w // PW
    g = (w % PW) // 16
    k = w % 16
    lo = g * 32 + k
    hi = lo + 16
    va = lo < D
    vb = hi < D
    ra = np.where(va, np.minimum(f, F - 1) * D + np.minimum(lo, D - 1), 0)
    rb = np.where(vb, np.minimum(f, F - 1) * D + np.minimum(hi, D - 1), 0)
    Wa = jnp.where(jnp.asarray(va)[:, None], W1[jnp.asarray(ra)], 0.0)
    Wb = jnp.where(jnp.asarray(vb)[:, None], W1[jnp.asarray(rb)], 0.0)
    return Wa.astype(jnp.bfloat16), Wb.astype(jnp.bfloat16)


def kernel(x_cat, emb, W1, b1, W2, b2, W3, b3, g2, be2, g3, be3):
    embp = jnp.pad(emb.astype(jnp.bfloat16), ((0, 0), (0, 0), (0, DK - D)))
    e = embp.reshape(F, V, 2, 2, 16)
    lo = lax.bitcast_convert_type(e[:, :, :, 0, :], jnp.uint16).astype(jnp.uint32)
    hi = lax.bitcast_convert_type(e[:, :, :, 1, :], jnp.uint16).astype(jnp.uint32)
    tpk = lax.bitcast_convert_type((hi << 16) | lo, jnp.int32).reshape(F * V * PW)
    Wa, Wb = _half_weights(W1)
    xct = jnp.transpose(x_cat.astype(jnp.int32)).reshape(F, B // CI, 1, CI)
    x = _gather(tpk, xct)
    return _mlp(x, Wa, Wb, b1.reshape(1, -1), W2, b2.reshape(1, -1),
                W3, b3.reshape(1, -1), g2.reshape(1, -1), be2.reshape(1, -1),
                g3.reshape(1, -1), be3.reshape(1, -1))


# speedup vs baseline: 1.3336x; 1.3336x over previous
import functools

import jax
import jax.numpy as jnp
import numpy as np
from jax import lax
from jax.experimental import pallas as pl
from jax.experimental.pallas import tpu as pltpu
from jax.experimental.pallas import tpu_sc as plsc

B = 16384
F = 26
V = 1000
D = 50
DP = 128
DK = 64
PW = DK // 2
ROW_W = F * PW
ROW_WP = 896
ROWS = B * F

NC = 2
NS = 16
NW = NC * NS
BW = B // NW
CB = 8
CH = CB * F
NITER = BW // CB


NUNIT = F * 4
QB = B // 4
CI = 512
NCH = QB // CI


def _gather_body(tpk_hbm, xct_hbm, out_hbm, tb, idx0, idx1, stg0, stg1,
                 ts, is0, is1, ws0, ws1):
    wid = lax.axis_index("s") * NC + lax.axis_index("c")

    def out_slice(f, i0):
        r0 = pl.multiple_of(f * PW, PW)
        c0 = pl.multiple_of(i0, CI)
        return out_hbm.at[pl.ds(r0, PW), pl.ds(c0, CI)]

    def do_chunk(f, cc, i0, idx_v, stg, isem, wsem, prev_f, prev_i0):
        pltpu.async_copy(xct_hbm.at[f, cc, 0], idx_v, isem).wait()

        @pl.when(prev_i0 >= 0)
        def _():
            pltpu.make_async_copy(stg, out_slice(prev_f, prev_i0), wsem).wait()

        @plsc.parallel_loop(0, CI // 16, unroll=4)
        def grp(j):
            rows = idx_v[pl.ds(j * 16, 16)]
            rbase = rows * PW
            for w in range(PW):
                vals = plsc.load_gather(tb, [rbase + w])
                stg[w, pl.ds(j * 16, 16)] = vals

        pltpu.async_copy(stg, out_slice(f, i0), wsem)

    def do_unit(s, carry):
        pf0, pi0, pf1, pi1 = carry
        u = wid + 32 * s
        f = u // 4
        q = u - 4 * f

        t0 = pl.multiple_of(f * (V * PW), V * PW)
        pltpu.async_copy(tpk_hbm.at[pl.ds(t0, V * PW)], tb, ts).wait()
        for c in range(NCH):
            cc = q * NCH + c
            i0 = cc * CI
            if c % 2 == 0:
                do_chunk(f, cc, i0, idx0, stg0, is0, ws0, pf0, pi0)
                pf0, pi0 = f, i0
            else:
                do_chunk(f, cc, i0, idx1, stg1, is1, ws1, pf1, pi1)
                pf1, pi1 = f, i0
        return pf0, pi0, pf1, pi1

    nu = jnp.where(wid < NUNIT - 3 * NW, 4, 3)
    carry = (jnp.int32(0), jnp.int32(-1), jnp.int32(0), jnp.int32(-1))
    pf0, pi0, pf1, pi1 = lax.fori_loop(0, nu, do_unit, carry, unroll=False)

    @pl.when(pi0 >= 0)
    def _():
        pltpu.make_async_copy(stg0, out_slice(pf0, pi0), ws0).wait()

    @pl.when(pi1 >= 0)
    def _():
        pltpu.make_async_copy(stg1, out_slice(pf1, pi1), ws1).wait()


@jax.jit
def _gather(tpk, xct):
    mesh = plsc.VectorSubcoreMesh(core_axis_name="c", subcore_axis_name="s")
    kern = pl.kernel(
        _gather_body,
        out_type=jax.ShapeDtypeStruct((ROW_W, B), jnp.int32),
        mesh=mesh,
        scratch_types=[
            pltpu.VMEM((V * PW,), jnp.int32),
            pltpu.VMEM((CI,), jnp.int32),
            pltpu.VMEM((CI,), jnp.int32),
            pltpu.VMEM((PW, CI), jnp.int32),
            pltpu.VMEM((PW, CI), jnp.int32),
            pltpu.SemaphoreType.DMA,
            pltpu.SemaphoreType.DMA,
            pltpu.SemaphoreType.DMA,
            pltpu.SemaphoreType.DMA,
            pltpu.SemaphoreType.DMA,
        ],
        compiler_params=pltpu.CompilerParams(needs_layout_passes=False),
    )
    return kern(tpk, xct)


_CHUNK = 512
_NSTEP = B // _CHUNK


def _mlp_body(x_ref, wa_ref, wb_ref, b1_ref, w2_ref, b2_ref, w3_ref, b3_ref,
              g2_ref, be2_ref, g3_ref, be3_ref, out_ref,
              h1_scr, s1_scr, ss1_scr):
    i = pl.program_id(0)

    @pl.when(i == 0)
    def _():
        s1_scr[...] = jnp.zeros_like(s1_scr)
        ss1_scr[...] = jnp.zeros_like(ss1_scr)

    xw = x_ref[...]
    xa = lax.bitcast_convert_type(
        lax.shift_left(xw, 16), jnp.float32).astype(jnp.bfloat16)
    xb = lax.bitcast_convert_type(
        jnp.bitwise_and(xw, jnp.int32(-65536)), jnp.float32).astype(jnp.bfloat16)
    dn = (((0,), (0,)), ((), ()))
    h = (lax.dot_general(xa, wa_ref[...], dn, preferred_element_type=jnp.float32)
         + lax.dot_general(xb, wb_ref[...], dn, preferred_element_type=jnp.float32))
    h = jnp.maximum(h + b1_ref[...], 0.0)
    h1_scr[pl.ds(i * _CHUNK, _CHUNK), :] = h
    s1_scr[...] += jnp.sum(h, axis=0, keepdims=True)
    ss1_scr[...] += jnp.sum(h * h, axis=0, keepdims=True)

    @pl.when(i == _NSTEP - 1)
    def _():
        m1 = s1_scr[...] * (1.0 / B)
        v1 = ss1_scr[...] * (1.0 / B) - m1 * m1
        a1 = g2_ref[...] * lax.rsqrt(v1 + 1e-5)
        c1 = be2_ref[...] - m1 * a1
        h1 = h1_scr[...] * a1 + c1
        h2 = jnp.dot(h1, w2_ref[...], preferred_element_type=jnp.float32)
        h2 = jnp.maximum(h2 + b2_ref[...], 0.0)
        m2 = jnp.mean(h2, axis=0, keepdims=True)
        v2 = jnp.mean(h2 * h2, axis=0, keepdims=True) - m2 * m2
        a2 = g3_ref[...] * lax.rsqrt(v2 + 1e-5)
        c2 = be3_ref[...] - m2 * a2
        h2n = h2 * a2 + c2
        z = jnp.dot(h2n, w3_ref[...], preferred_element_type=jnp.float32)
        z = z + b3_ref[...]
        out_ref[...] = jax.nn.sigmoid(z)


@jax.jit
def _mlp(x, Wa, Wb, b1, W2, b2, W3, b3, g2, be2, g3, be3):
    const = lambda shape: pl.BlockSpec(shape, lambda i: (0, 0))
    return pl.pallas_call(
        _mlp_body,
        grid=(_NSTEP,),
        in_specs=[
            pl.BlockSpec((ROW_W, _CHUNK), lambda i: (0, i)),
            const((ROW_W, 150)),
            const((ROW_W, 150)),
            const((1, 150)),
            const((150, 75)),
            const((1, 75)),
            const((75, 1)),
            const((1, 1)),
            const((1, 150)),
            const((1, 150)),
            const((1, 75)),
            const((1, 75)),
        ],
        out_specs=pl.BlockSpec((B, 1), lambda i: (0, 0)),
        out_shape=jax.ShapeDtypeStruct((B, 1), jnp.float32),
        scratch_shapes=[
            pltpu.VMEM((B, 150), jnp.float32),
            pltpu.VMEM((1, 150), jnp.float32),
            pltpu.VMEM((1, 150), jnp.float32),
        ],
        compiler_params=pltpu.CompilerParams(
            dimension_semantics=("arbitrary",),
            vmem_limit_bytes=100 * 1024 * 1024,
        ),
    )(x, Wa, Wb, b1, W2, b2, W3, b3, g2, be2, g3, be3)


def _half_weights(W1):
    w = np.arange(ROW_W)
    f = w // PW
    g = (w % PW) // 16
    k = w % 16
    lo = g * 32 + k
    hi = lo + 16
    va = lo < D
    vb = hi < D
    ra = np.where(va, np.minimum(f, F - 1) * D + np.minimum(lo, D - 1), 0)
    rb = np.where(vb, np.minimum(f, F - 1) * D + np.minimum(hi, D - 1), 0)
    Wa = jnp.where(jnp.asarray(va)[:, None], W1[jnp.asarray(ra)], 0.0)
    Wb = jnp.where(jnp.asarray(vb)[:, None], W1[jnp.asarray(rb)], 0.0)
    return Wa.astype(jnp.bfloat16), Wb.astype(jnp.bfloat16)


def kernel(x_cat, emb, W1, b1, W2, b2, W3, b3, g2, be2, g3, be3):
    embp = jnp.pad(emb.astype(jnp.bfloat16), ((0, 0), (0, 0), (0, DK - D)))
    e = embp.reshape(F, V, 2, 2, 16)
    lo = lax.bitcast_convert_type(e[:, :, :, 0, :], jnp.uint16).astype(jnp.uint32)
    hi = lax.bitcast_convert_type(e[:, :, :, 1, :], jnp.uint16).astype(jnp.uint32)
    tpk = lax.bitcast_convert_type((hi << 16) | lo, jnp.int32).reshape(F * V * PW)
    Wa, Wb = _half_weights(W1)
    xct = jnp.transpose(x_cat.astype(jnp.int32)).reshape(F, B // CI, 1, CI)
    x = _gather(tpk, xct)
    return _mlp(x, Wa, Wb, b1.reshape(1, -1), W2, b2.reshape(1, -1),
                W3, b3.reshape(1, -1), g2.reshape(1, -1), be2.reshape(1, -1),
                g3.reshape(1, -1), be3.reshape(1, -1))


# speedup vs baseline: 2.2350x; 1.6759x over previous
import functools

import jax
import jax.numpy as jnp
import numpy as np
from jax import lax
from jax.experimental import pallas as pl
from jax.experimental.pallas import tpu as pltpu
from jax.experimental.pallas import tpu_sc as plsc

B = 16384
F = 26
V = 1000
D = 50
DP = 128
DK = 64
PW = DK // 2
ROW_W = F * PW
ROW_WP = 896
ROWS = B * F

NC = 2
NS = 16
NW = NC * NS
S = 4
BS = B // S
BW = BS // NW
CB = 8
CH = CB * F
NITER = BW // CB


def _pack_rows(buf, cbuf, r8):
    for f in range(F):
        row = r8 * F + f
        for g in range(2):
            a = buf[row, pl.ds(g * 32, 16)]
            b = buf[row, pl.ds(g * 32 + 16, 16)]
            p = plsc.pack(a, b, format=plsc.PackFormat.INTERLEAVED)
            w = plsc.bitcast(p, jnp.int32)
            cbuf[r8, pl.ds(f * PW + g * 16, 16)] = w


def _gather_body(emb_hbm, idx_hbm, out_hbm, idx_v0, idx_v1, buf0, buf1,
                 cbuf0, cbuf1, gs0, gs1, ws0, ws1):
    wid = lax.axis_index("s") * NC + lax.axis_index("c")
    base = wid * BW

    zero = jnp.zeros((16,), jnp.int32)
    for cbuf in (cbuf0, cbuf1):
        for r8 in range(CB):
            for w in range((ROW_WP - ROW_W) // 16):
                cbuf[r8, pl.ds(ROW_W + w * 16, 16)] = zero

    def fire(c, idx_v, buf, sem):
        pltpu.sync_copy(idx_hbm.at[wid, c], idx_v)
        pltpu.async_copy(emb_hbm.at[idx_v], buf, sem)

    def wait_gather(idx_v, buf, sem):
        pltpu.make_async_copy(emb_hbm.at[idx_v], buf, sem).wait()

    def put(c, cbuf, sem):
        pltpu.async_copy(cbuf, out_hbm.at[pl.ds((base + c * CB), CB)], sem)

    def wait_put(c, cbuf, sem):
        pltpu.make_async_copy(cbuf, out_hbm.at[pl.ds((base + c * CB), CB)],
                              sem).wait()

    fire(0, idx_v0, buf0, gs0)

    def body(j, carry):
        c0 = 2 * j
        fire(c0 + 1, idx_v1, buf1, gs1)
        wait_gather(idx_v0, buf0, gs0)

        @pl.when(j > 0)
        def _():
            wait_put(c0 - 2, cbuf0, ws0)

        def pk0(r8, carry):
            _pack_rows(buf0, cbuf0, r8)
            return carry
        lax.fori_loop(0, CB, pk0, 0, unroll=False)
        put(c0, cbuf0, ws0)

        @pl.when(j < NITER // 2 - 1)
        def _():
            fire(c0 + 2, idx_v0, buf0, gs0)

        wait_gather(idx_v1, buf1, gs1)

        @pl.when(j > 0)
        def _():
            wait_put(c0 - 1, cbuf1, ws1)

        def pk1(r8, carry):
            _pack_rows(buf1, cbuf1, r8)
            return carry
        lax.fori_loop(0, CB, pk1, 0, unroll=False)
        put(c0 + 1, cbuf1, ws1)
        return carry

    lax.fori_loop(0, NITER // 2, body, 0, unroll=False)
    wait_put(NITER - 2, cbuf0, ws0)
    wait_put(NITER - 1, cbuf1, ws1)


@jax.jit
def _gather(emb_flat, idx):
    mesh = plsc.VectorSubcoreMesh(core_axis_name="c", subcore_axis_name="s")
    kern = pl.kernel(
        _gather_body,
        out_type=jax.ShapeDtypeStruct((BS, ROW_WP), jnp.int32),
        mesh=mesh,
        scratch_types=[
            pltpu.VMEM((CH,), jnp.int32),
            pltpu.VMEM((CH,), jnp.int32),
            pltpu.VMEM((CH, DP), jnp.float32),
            pltpu.VMEM((CH, DP), jnp.float32),
            pltpu.VMEM((CB, ROW_WP), jnp.int32),
            pltpu.VMEM((CB, ROW_WP), jnp.int32),
            pltpu.SemaphoreType.DMA,
            pltpu.SemaphoreType.DMA,
            pltpu.SemaphoreType.DMA,
            pltpu.SemaphoreType.DMA,
        ],
        compiler_params=pltpu.CompilerParams(needs_layout_passes=False),
    )
    return kern(emb_flat, idx)


_CHUNK = 512
_NSTEP1 = BS // _CHUNK


def _mlp1_body(x_ref, wa_ref, wb_ref, b1_ref, h1_ref, st_ref,
               s1_scr, ss1_scr):
    i = pl.program_id(0)

    @pl.when(i == 0)
    def _():
        s1_scr[...] = jnp.zeros_like(s1_scr)
        ss1_scr[...] = jnp.zeros_like(ss1_scr)

    xw = x_ref[...]
    xa = lax.bitcast_convert_type(
        lax.shift_left(xw, 16), jnp.float32).astype(jnp.bfloat16)
    xb = lax.bitcast_convert_type(
        jnp.bitwise_and(xw, jnp.int32(-65536)), jnp.float32).astype(jnp.bfloat16)
    h = (jnp.dot(xa, wa_ref[...], preferred_element_type=jnp.float32)
         + jnp.dot(xb, wb_ref[...], preferred_element_type=jnp.float32))
    h = jnp.maximum(h + b1_ref[...], 0.0)
    h1_ref[...] = h
    s1_scr[...] += jnp.sum(h, axis=0, keepdims=True)
    ss1_scr[...] += jnp.sum(h * h, axis=0, keepdims=True)

    @pl.when(i == _NSTEP1 - 1)
    def _():
        st_ref[0:1, :] = s1_scr[...]
        st_ref[1:2, :] = ss1_scr[...]


@jax.jit
def _mlp1(x, Wa, Wb, b1):
    const = lambda shape: pl.BlockSpec(shape, lambda i: (0, 0))
    return pl.pallas_call(
        _mlp1_body,
        grid=(_NSTEP1,),
        in_specs=[
            pl.BlockSpec((_CHUNK, ROW_WP), lambda i: (i, 0)),
            const((ROW_WP, 150)),
            const((ROW_WP, 150)),
            const((1, 150)),
        ],
        out_specs=[
            pl.BlockSpec((_CHUNK, 150), lambda i: (i, 0)),
            pl.BlockSpec((2, 150), lambda i: (0, 0)),
        ],
        out_shape=[
            jax.ShapeDtypeStruct((BS, 150), jnp.float32),
            jax.ShapeDtypeStruct((2, 150), jnp.float32),
        ],
        scratch_shapes=[
            pltpu.VMEM((1, 150), jnp.float32),
            pltpu.VMEM((1, 150), jnp.float32),
        ],
        compiler_params=pltpu.CompilerParams(
            dimension_semantics=("arbitrary",),
            vmem_limit_bytes=100 * 1024 * 1024,
        ),
    )(x, Wa, Wb, b1)


def _mlp2_body(h1a_ref, h1b_ref, h1c_ref, h1d_ref, st_ref,
               w2_ref, b2_ref, w3_ref, b3_ref,
               g2_ref, be2_ref, g3_ref, be3_ref, out_ref, h2_scr):
    h1_refs = (h1a_ref, h1b_ref, h1c_ref, h1d_ref)
    st = st_ref[...]
    s1 = st[0:1] + st[2:3] + st[4:5] + st[6:7]
    ss1 = st[1:2] + st[3:4] + st[5:6] + st[7:8]
    m1 = s1 * (1.0 / B)
    v1 = ss1 * (1.0 / B) - m1 * m1
    a1 = g2_ref[...] * lax.rsqrt(v1 + 1e-5)
    c1 = be2_ref[...] - m1 * a1
    s2 = jnp.zeros((1, 75), jnp.float32)
    ss2 = jnp.zeros((1, 75), jnp.float32)
    for s in range(S):
        h1n = h1_refs[s][...] * a1 + c1
        h2 = jnp.dot(h1n, w2_ref[...], preferred_element_type=jnp.float32)
        h2 = jnp.maximum(h2 + b2_ref[...], 0.0)
        h2_scr[pl.ds(s * BS, BS), :] = h2
        s2 = s2 + jnp.sum(h2, axis=0, keepdims=True)
        ss2 = ss2 + jnp.sum(h2 * h2, axis=0, keepdims=True)
    m2 = s2 * (1.0 / B)
    v2 = ss2 * (1.0 / B) - m2 * m2
    a2 = g3_ref[...] * lax.rsqrt(v2 + 1e-5)
    c2 = be3_ref[...] - m2 * a2
    for s in range(S):
        h2n = h2_scr[pl.ds(s * BS, BS), :] * a2 + c2
        z = jnp.dot(h2n, w3_ref[...], preferred_element_type=jnp.float32)
        z = z + b3_ref[...]
        out_ref[pl.ds(s * BS, BS), :] = jax.nn.sigmoid(z)


@jax.jit
def _mlp2(h1s, st, W2, b2, W3, b3, g2, be2, g3, be3):
    const = lambda shape: pl.BlockSpec(shape, lambda i: (0, 0))
    return pl.pallas_call(
        _mlp2_body,
        grid=(1,),
        in_specs=[
            const((BS, 150)),
            const((BS, 150)),
            const((BS, 150)),
            const((BS, 150)),
            const((2 * S, 150)),
            const((150, 75)),
            const((1, 75)),
            const((75, 1)),
            const((1, 1)),
            const((1, 150)),
            const((1, 150)),
            const((1, 75)),
            const((1, 75)),
        ],
        out_specs=pl.BlockSpec((B, 1), lambda i: (0, 0)),
        out_shape=jax.ShapeDtypeStruct((B, 1), jnp.float32),
        scratch_shapes=[
            pltpu.VMEM((B, 75), jnp.float32),
        ],
        compiler_params=pltpu.CompilerParams(
            vmem_limit_bytes=100 * 1024 * 1024,
        ),
    )(*h1s, st, W2, b2, W3, b3, g2, be2, g3, be3)


def _half_weights(W1):
    w = np.arange(ROW_WP)
    f = w // PW
    g = (w % PW) // 16
    k = w % 16
    lo = g * 32 + k
    hi = lo + 16
    valid = w < ROW_W
    va = valid & (lo < D)
    vb = valid & (hi < D)
    ra = np.where(va, np.minimum(f, F - 1) * D + np.minimum(lo, D - 1), 0)
    rb = np.where(vb, np.minimum(f, F - 1) * D + np.minimum(hi, D - 1), 0)
    Wa = jnp.where(jnp.asarray(va)[:, None], W1[jnp.asarray(ra)], 0.0)
    Wb = jnp.where(jnp.asarray(vb)[:, None], W1[jnp.asarray(rb)], 0.0)
    return Wa.astype(jnp.bfloat16), Wb.astype(jnp.bfloat16)


def kernel(x_cat, emb, W1, b1, W2, b2, W3, b3, g2, be2, g3, be3):
    emb_flat = jnp.pad(emb.reshape(F * V, D), ((0, 0), (0, DP - D)))
    Wa, Wb = _half_weights(W1)
    offs = (jnp.arange(F, dtype=jnp.int32) * V)[None, :]
    idx = (x_cat.astype(jnp.int32) + offs).reshape(S, NW, NITER, CH)
    h1s, sts = [], []
    for s in range(S):
        x_s = _gather(emb_flat, idx[s])
        h1_s, st_s = _mlp1(x_s, Wa, Wb, b1.reshape(1, -1))
        h1s.append(h1_s)
        sts.append(st_s)
    st = jnp.concatenate(sts, axis=0)
    return _mlp2(h1s, st, W2, b2.reshape(1, -1),
                 W3, b3.reshape(1, -1), g2.reshape(1, -1), be2.reshape(1, -1),
                 g3.reshape(1, -1), be3.reshape(1, -1))
